# baseline (speedup 1.0000x reference)
import functools

import jax
import jax.numpy as jnp
from jax import lax
from jax.experimental import pallas as pl
from jax.experimental.pallas import tpu as pltpu
from jax.experimental.pallas import tpu_sc as plsc

N = 10000
E = 320000
D = 128
G = 64
C = 10

NC = 2
NS = 16

DEG_E = 327680
DEG_PAD = DEG_E - E
DEG_ACC = 10240
DEG_TILE_E = DEG_E // (NC * NS)

EP = 327680
EDGES_PER_TILE = EP // (NC * NS)
CHUNK_E = 128
N_CHUNKS = EDGES_PER_TILE // CHUNK_E
GROUP = 8
N_GROUPS = N_CHUNKS // GROUP
NPAD = 10240
ZROWS = 128


def _deg_body(dstp_hbm, degp_hbm, idx_v, vals_v, zsrc_v, acc_sh):
  c = lax.axis_index("c")
  s = lax.axis_index("s")

  @pl.loop(0, (DEG_ACC // NS) // 16)
  def _(i):
    zsrc_v[pl.ds(i * 16, 16)] = jnp.zeros((16,), jnp.float32)

  pltpu.sync_copy(zsrc_v, acc_sh.at[pl.ds(s * (DEG_ACC // NS), DEG_ACC // NS)])
  plsc.subcore_barrier()

  @pl.loop(0, DEG_TILE_E // 16)
  def _(i):
    vals_v[pl.ds(i * 16, 16)] = jnp.full((16,), 1.0, jnp.float32)

  e0 = c * (DEG_E // NC) + s * DEG_TILE_E
  pltpu.sync_copy(dstp_hbm.at[pl.ds(e0, DEG_TILE_E)], idx_v)
  pltpu.sync_copy(vals_v, acc_sh.at[idx_v], add=True)
  plsc.subcore_barrier()

  @pl.when(s == 0)
  def _():
    pltpu.sync_copy(acc_sh, degp_hbm.at[pl.ds(c * DEG_ACC, DEG_ACC)])


def _agg_body(src_hbm, dst_hbm, hs_hbm, outp_hbm, sidx_v, didx_v, rows_v,
              isems, gsems, ssems, acc_sh):
  c = lax.axis_index("c")
  s = lax.axis_index("s")
  base_e = (c * NS + s) * EDGES_PER_TILE

  base_r = base_e // CHUNK_E

  def issue_group(g_expr, b):
    r0 = pl.multiple_of(base_r + g_expr * GROUP, 8)
    pltpu.async_copy(src_hbm.at[pl.ds(r0, GROUP)], sidx_v.at[b], isems.at[b])
    pltpu.async_copy(dst_hbm.at[pl.ds(r0, GROUP)], didx_v.at[b], isems.at[b])

  def wait_group(b):
    pltpu.make_async_copy(src_hbm.at[pl.ds(0, GROUP)], sidx_v.at[b],
                          isems.at[b]).wait()
    pltpu.make_async_copy(dst_hbm.at[pl.ds(0, GROUP)], didx_v.at[b],
                          isems.at[b]).wait()

  def gather(r, b, jj):
    pltpu.async_copy(hs_hbm.at[sidx_v.at[b, jj]], rows_v.at[r], gsems.at[r])

  def wait_gather(r, b, jj):
    pltpu.make_async_copy(hs_hbm.at[sidx_v.at[b, jj]], rows_v.at[r],
                          gsems.at[r]).wait()

  def scatter(r, b, jj):
    pltpu.async_copy(rows_v.at[r], acc_sh.at[didx_v.at[b, jj]], ssems.at[r],
                     add=True)

  def wait_scatter(r, b, jj):
    pltpu.make_async_copy(rows_v.at[r], acc_sh.at[didx_v.at[b, jj]],
                          ssems.at[r]).wait()

  issue_group(0, 0)
  wait_group(0)
  gather(0, 0, 0)

  @pl.loop(0, ZROWS * 8)
  def _(i):
    rows_v[1, i // 8, pl.ds((i % 8) * 16, 16)] = jnp.zeros((16,), jnp.float32)

  @pl.loop(0, 640 // ZROWS)
  def _(i):
    pltpu.sync_copy(rows_v.at[1, pl.ds(0, ZROWS)],
                    acc_sh.at[pl.ds(s * 640 + i * ZROWS, ZROWS)])

  plsc.subcore_barrier()

  @pl.loop(0, N_GROUPS // 2)
  def _(u):
    not_last_u = u < N_GROUPS // 2 - 1

    def group_body(g_expr, bg, first_guard, prefetch_guard, next_guard):
      for jj in range(GROUP):
        r = jj % 2
        pr, pb, pjj = (jj + 1) % 2, (bg if jj > 0 else 1 - bg), (jj - 1) % GROUP
        if jj == 0 and first_guard is not None:
          @pl.when(first_guard)
          def _():
            wait_scatter(pr, pb, pjj)
        else:
          wait_scatter(pr, pb, pjj)

        if jj == 0:
          if prefetch_guard is None:
            issue_group(g_expr + 1, 1 - bg)
          else:
            @pl.when(prefetch_guard)
            def _():
              issue_group(g_expr + 1, 1 - bg)

        wait_gather(r, bg, jj)
        scatter(r, bg, jj)

        if jj < GROUP - 1:
          gather((jj + 1) % 2, bg, jj + 1)
        elif next_guard is None:
          wait_group(1 - bg)
          gather((jj + 1) % 2, 1 - bg, 0)
        else:
          @pl.when(next_guard)
          def _():
            wait_group(1 - bg)
            gather((jj + 1) % 2, 1 - bg, 0)

    g0 = 2 * u
    group_body(g0, 0, first_guard=(u > 0), prefetch_guard=None,
               next_guard=None)
    group_body(g0 + 1, 1, first_guard=None, prefetch_guard=not_last_u,
               next_guard=not_last_u)

  wait_scatter(1, 1, GROUP - 1)
  plsc.subcore_barrier()
  pltpu.sync_copy(acc_sh.at[pl.ds(s * 640, 640)],
                  outp_hbm.at[c, pl.ds(s * 640, 640)])


def _sc_mesh():
  return plsc.VectorSubcoreMesh(
      core_axis_name="c", subcore_axis_name="s", num_cores=NC, num_subcores=NS
  )


def _sc_degrees(dstp):
  k = pl.kernel(
      _deg_body,
      out_type=jax.ShapeDtypeStruct((NC * DEG_ACC,), jnp.float32),
      mesh=_sc_mesh(),
      scratch_types=[
          pltpu.VMEM((DEG_TILE_E,), jnp.int32),
          pltpu.VMEM((DEG_TILE_E,), jnp.float32),
          pltpu.VMEM((DEG_ACC // NS,), jnp.float32),
          pltpu.VMEM_SHARED((DEG_ACC,), jnp.float32),
      ],
  )
  return k(dstp)


def _sc_aggregate(src, dst, hs):
  k = pl.kernel(
      _agg_body,
      out_type=jax.ShapeDtypeStruct((NC, NPAD, D), jnp.float32),
      mesh=_sc_mesh(),
      scratch_types=[
          pltpu.VMEM((2, GROUP, CHUNK_E), jnp.int32),
          pltpu.VMEM((2, GROUP, CHUNK_E), jnp.int32),
          pltpu.VMEM((2, CHUNK_E, D), jnp.float32),
          pltpu.SemaphoreType.DMA((2,)),
          pltpu.SemaphoreType.DMA((2,)),
          pltpu.SemaphoreType.DMA((2,)),
          pltpu.VMEM_SHARED((NPAD, D), jnp.float32),
      ],
  )
  return k(src, dst, hs)


def _dot(a, b):
  return lax.dot_general(
      a, b, (((1,), (0,)), ((), ())),
      precision=lax.Precision.HIGHEST,
      preferred_element_type=jnp.float32,
  )


NB = 5
BR = N // NB


def _first_body(degt_ref, x_ref, w_ref, dinv_ref, hs_ref):
  deg = degt_ref[:, 0:1] + degt_ref[:, 1:2] + 1.0
  dinv = lax.rsqrt(deg)
  dinv_ref[...] = dinv
  hs_ref[...] = _dot(x_ref[...], w_ref[...]) * dinv


def _first_call(degt, x, W1):
  return pl.pallas_call(
      _first_body,
      grid=(NB,),
      in_specs=[
          pl.BlockSpec((BR, 2), lambda i: (i, 0)),
          pl.BlockSpec((BR, D), lambda i: (i, 0)),
          pl.BlockSpec((D, D), lambda i: (0, 0)),
      ],
      out_specs=[
          pl.BlockSpec((BR, 1), lambda i: (i, 0)),
          pl.BlockSpec((BR, D), lambda i: (i, 0)),
      ],
      out_shape=(jax.ShapeDtypeStruct((N, 1), jnp.float32),
                 jax.ShapeDtypeStruct((N, D), jnp.float32)),
  )(degt, x, W1)


def _mid_body(aggp_ref, hs_ref, dinv_ref, b_ref, g_ref, be_ref, w_ref,
              hs2_ref, a_scr, st_scr, *, relu):
  p = pl.program_id(0)
  i = pl.program_id(1)
  rows = pl.ds(i * BR, BR)

  @pl.when(p == 0)
  def _():
    agg = aggp_ref[0] + aggp_ref[1] + hs_ref[...]
    conv = dinv_ref[...] * agg + b_ref[...]
    a = jnp.maximum(conv, 0.0) if relu else conv
    a_scr[rows, :] = a

    @pl.when(i == 0)
    def _():
      st_scr[...] = jnp.zeros((2, D), jnp.float32)

    st_scr[0:1, :] += jnp.sum(a, axis=0, keepdims=True)
    st_scr[1:2, :] += jnp.sum(a * a, axis=0, keepdims=True)

  @pl.when(p == 1)
  def _():
    m = st_scr[0:1, :] * (1.0 / N)
    v = st_scr[1:2, :] * (1.0 / N) - m * m
    a = a_scr[rows, :]
    z = (a - m) * lax.rsqrt(v + 1e-5) * g_ref[...] + be_ref[...]
    hs2_ref[...] = _dot(z, w_ref[...]) * dinv_ref[...]


def _mid_call(aggp, hs, dinv, br, gr, ber, W, relu):
  return pl.pallas_call(
      functools.partial(_mid_body, relu=relu),
      grid=(2, NB),
      in_specs=[
          pl.BlockSpec((2, BR, D), lambda p, i: (0, jnp.where(p == 0, i, 0), 0)),
          pl.BlockSpec((BR, D), lambda p, i: (jnp.where(p == 0, i, 0), 0)),
          pl.BlockSpec((BR, 1), lambda p, i: (i, 0)),
          pl.BlockSpec((1, D), lambda p, i: (0, 0)),
          pl.BlockSpec((1, D), lambda p, i: (0, 0)),
          pl.BlockSpec((1, D), lambda p, i: (0, 0)),
          pl.BlockSpec((D, D), lambda p, i: (0, 0)),
      ],
      out_specs=pl.BlockSpec((BR, D), lambda p, i: (i, 0)),
      out_shape=jax.ShapeDtypeStruct((N, D), jnp.float32),
      scratch_shapes=[
          pltpu.VMEM((N, D), jnp.float32),
          pltpu.VMEM((2, D), jnp.float32),
      ],
  )(aggp, hs, dinv, br, gr, ber, W)


def _final_body(aggp_ref, hs_ref, dinv_ref, b_ref, g_ref, be_ref,
                batch_ref, linw_ref, linb_ref, out_ref, a_scr, st_scr,
                ps_scr, cnt_scr):
  p = pl.program_id(0)
  i = pl.program_id(1)
  rows = pl.ds(i * BR, BR)

  @pl.when(p == 0)
  def _():
    agg = aggp_ref[0] + aggp_ref[1] + hs_ref[...]
    conv = dinv_ref[...] * agg + b_ref[...]
    a_scr[rows, :] = conv

    @pl.when(i == 0)
    def _():
      st_scr[...] = jnp.zeros((2, D), jnp.float32)

    st_scr[0:1, :] += jnp.sum(conv, axis=0, keepdims=True)
    st_scr[1:2, :] += jnp.sum(conv * conv, axis=0, keepdims=True)

  @pl.when(p == 1)
  def _():
    m = st_scr[0:1, :] * (1.0 / N)
    v = st_scr[1:2, :] * (1.0 / N) - m * m
    a = a_scr[rows, :]
    z = (a - m) * lax.rsqrt(v + 1e-5) * g_ref[...] + be_ref[...]
    seg = lax.broadcasted_iota(jnp.int32, (G, BR), 0)
    bt = batch_ref[pl.ds(i, 1), :]
    pm = (seg == jnp.broadcast_to(bt, (G, BR))).astype(jnp.float32)

    @pl.when(i == 0)
    def _():
      ps_scr[...] = jnp.zeros((G, D), jnp.float32)
      cnt_scr[...] = jnp.zeros((G, 1), jnp.float32)

    ps_scr[...] += _dot(pm, z)
    cnt_scr[...] += jnp.sum(pm, axis=1, keepdims=True)

    @pl.when(i == NB - 1)
    def _():
      pooled = ps_scr[...] / jnp.maximum(cnt_scr[...], 1.0)
      out_ref[...] = _dot(pooled, linw_ref[...]) + linb_ref[...]


def _final_call(aggp, hs, dinv, br, gr, ber, batchr, linW, linbr):
  return pl.pallas_call(
      _final_body,
      grid=(2, NB),
      in_specs=[
          pl.BlockSpec((2, BR, D), lambda p, i: (0, jnp.where(p == 0, i, 0), 0)),
          pl.BlockSpec((BR, D), lambda p, i: (jnp.where(p == 0, i, 0), 0)),
          pl.BlockSpec((BR, 1), lambda p, i: (i, 0)),
          pl.BlockSpec((1, D), lambda p, i: (0, 0)),
          pl.BlockSpec((1, D), lambda p, i: (0, 0)),
          pl.BlockSpec((1, D), lambda p, i: (0, 0)),
          pl.BlockSpec((NB, BR), lambda p, i: (0, 0)),
          pl.BlockSpec((D, C), lambda p, i: (0, 0)),
          pl.BlockSpec((1, C), lambda p, i: (0, 0)),
      ],
      out_specs=pl.BlockSpec((G, C), lambda p, i: (0, 0)),
      out_shape=jax.ShapeDtypeStruct((G, C), jnp.float32),
      scratch_shapes=[
          pltpu.VMEM((N, D), jnp.float32),
          pltpu.VMEM((2, D), jnp.float32),
          pltpu.VMEM((G, D), jnp.float32),
          pltpu.VMEM((G, 1), jnp.float32),
      ],
  )(aggp, hs, dinv, br, gr, ber, batchr, linW, linbr)


def kernel(x, edge_index, batch, hidden_channels, num_layers,
           W1, b1, g1, be1, W2, b2, g2, be2, W3, b3, g3, be3, linW, linb):
  del hidden_channels, num_layers
  src = edge_index[0]
  dst = edge_index[1]

  padi = jnp.arange(DEG_PAD, dtype=jnp.int32) % (DEG_ACC - N)
  dstp = jnp.concatenate([dst, N + padi])
  src2d = jnp.concatenate(
      [src.reshape(E // 128, 128), padi.reshape(DEG_PAD // 128, 128)])
  dst2d = jnp.concatenate(
      [dst.reshape(E // 128, 128), (N + padi).reshape(DEG_PAD // 128, 128)])

  degp = _sc_degrees(dstp).reshape(NC, DEG_ACC)
  degt = degp[:, :N].T

  dinv, hs1 = _first_call(degt, x, W1)

  b1r, g1r, be1r = b1.reshape(1, D), g1.reshape(1, D), be1.reshape(1, D)
  b2r, g2r, be2r = b2.reshape(1, D), g2.reshape(1, D), be2.reshape(1, D)
  b3r, g3r, be3r = b3.reshape(1, D), g3.reshape(1, D), be3.reshape(1, D)

  agg1p = _sc_aggregate(src2d, dst2d, hs1)
  hs2 = _mid_call(agg1p, hs1, dinv, b1r, g1r, be1r, W2, True)

  agg2p = _sc_aggregate(src2d, dst2d, hs2)
  hs3 = _mid_call(agg2p, hs2, dinv, b2r, g2r, be2r, W3, True)

  agg3p = _sc_aggregate(src2d, dst2d, hs3)
  return _final_call(agg3p, hs3, dinv, b3r, g3r, be3r,
                     batch.reshape(NB, BR), linW, linb.reshape(1, C))


# speedup vs baseline: 1.0069x; 1.0069x over previous
import functools

import jax
import jax.numpy as jnp
from jax import lax
from jax.experimental import pallas as pl
from jax.experimental.pallas import tpu as pltpu
from jax.experimental.pallas import tpu_sc as plsc

N = 10000
E = 320000
D = 128
G = 64
C = 10

NC = 2
NS = 16

DEG_E = 327680
DEG_PAD = DEG_E - E
DEG_ACC = 10240
DEG_TILE_E = DEG_E // (NC * NS)

EP = 327680
EDGES_PER_TILE = EP // (NC * NS)
CHUNK_E = 128
N_CHUNKS = EDGES_PER_TILE // CHUNK_E
GROUP = 8
N_GROUPS = N_CHUNKS // GROUP
NPAD = 10240
ZROWS = 128


def _deg_body(dstp_hbm, degp_hbm, idx_v, vals_v, zsrc_v, acc_sh):
  c = lax.axis_index("c")
  s = lax.axis_index("s")

  @pl.loop(0, (DEG_ACC // NS) // 16)
  def _(i):
    zsrc_v[pl.ds(i * 16, 16)] = jnp.zeros((16,), jnp.float32)

  pltpu.sync_copy(zsrc_v, acc_sh.at[pl.ds(s * (DEG_ACC // NS), DEG_ACC // NS)])
  plsc.subcore_barrier()

  @pl.loop(0, DEG_TILE_E // 16)
  def _(i):
    vals_v[pl.ds(i * 16, 16)] = jnp.full((16,), 1.0, jnp.float32)

  e0 = c * (DEG_E // NC) + s * DEG_TILE_E
  pltpu.sync_copy(dstp_hbm.at[pl.ds(e0, DEG_TILE_E)], idx_v)
  pltpu.sync_copy(vals_v, acc_sh.at[idx_v], add=True)
  plsc.subcore_barrier()

  @pl.when(s == 0)
  def _():
    pltpu.sync_copy(acc_sh, degp_hbm.at[pl.ds(c * DEG_ACC, DEG_ACC)])


def _agg_body(src_hbm, dst_hbm, hs_hbm, outp_hbm, sidx_v, didx_v, rows_v,
              isems, gsems, ssems, acc_sh):
  c = lax.axis_index("c")
  s = lax.axis_index("s")
  base_e = (c * NS + s) * EDGES_PER_TILE

  base_r = base_e // CHUNK_E

  def issue_group(g_expr, b):
    r0 = pl.multiple_of(base_r + g_expr * GROUP, 8)
    pltpu.async_copy(src_hbm.at[pl.ds(r0, GROUP)], sidx_v.at[b], isems.at[b])
    pltpu.async_copy(dst_hbm.at[pl.ds(r0, GROUP)], didx_v.at[b], isems.at[b])

  def wait_group(b):
    pltpu.make_async_copy(src_hbm.at[pl.ds(0, GROUP)], sidx_v.at[b],
                          isems.at[b]).wait()
    pltpu.make_async_copy(dst_hbm.at[pl.ds(0, GROUP)], didx_v.at[b],
                          isems.at[b]).wait()

  def gather(r, b, jj):
    pltpu.async_copy(hs_hbm.at[sidx_v.at[b, jj]], rows_v.at[r], gsems.at[r])

  def wait_gather(r, b, jj):
    pltpu.make_async_copy(hs_hbm.at[sidx_v.at[b, jj]], rows_v.at[r],
                          gsems.at[r]).wait()

  def scatter(r, b, jj):
    pltpu.async_copy(rows_v.at[r], acc_sh.at[didx_v.at[b, jj]], ssems.at[r],
                     add=True)

  def wait_scatter(r, b, jj):
    pltpu.make_async_copy(rows_v.at[r], acc_sh.at[didx_v.at[b, jj]],
                          ssems.at[r]).wait()

  issue_group(0, 0)
  wait_group(0)
  gather(0, 0, 0)

  @pl.loop(0, ZROWS * 8)
  def _(i):
    rows_v[1, i // 8, pl.ds((i % 8) * 16, 16)] = jnp.zeros((16,), jnp.float32)

  @pl.loop(0, 640 // ZROWS)
  def _(i):
    pltpu.sync_copy(rows_v.at[1, pl.ds(0, ZROWS)],
                    acc_sh.at[pl.ds(s * 640 + i * ZROWS, ZROWS)])

  plsc.subcore_barrier()

  @pl.loop(0, N_GROUPS // 2)
  def _(u):
    not_last_u = u < N_GROUPS // 2 - 1

    def group_body(g_expr, bg, first_guard, prefetch_guard, next_guard):
      for jj in range(GROUP):
        r = jj % 2
        pr, pb, pjj = (jj + 1) % 2, (bg if jj > 0 else 1 - bg), (jj - 1) % GROUP
        if jj == 0 and first_guard is not None:
          @pl.when(first_guard)
          def _():
            wait_scatter(pr, pb, pjj)
        else:
          wait_scatter(pr, pb, pjj)

        if jj == 0:
          if prefetch_guard is None:
            issue_group(g_expr + 1, 1 - bg)
          else:
            @pl.when(prefetch_guard)
            def _():
              issue_group(g_expr + 1, 1 - bg)

        wait_gather(r, bg, jj)
        scatter(r, bg, jj)

        if jj < GROUP - 1:
          gather((jj + 1) % 2, bg, jj + 1)
        elif next_guard is None:
          wait_group(1 - bg)
          gather((jj + 1) % 2, 1 - bg, 0)
        else:
          @pl.when(next_guard)
          def _():
            wait_group(1 - bg)
            gather((jj + 1) % 2, 1 - bg, 0)

    g0 = 2 * u
    group_body(g0, 0, first_guard=(u > 0), prefetch_guard=None,
               next_guard=None)
    group_body(g0 + 1, 1, first_guard=None, prefetch_guard=not_last_u,
               next_guard=not_last_u)

  wait_scatter(1, 1, GROUP - 1)
  plsc.subcore_barrier()
  pltpu.sync_copy(acc_sh.at[pl.ds(s * 640, 640)],
                  outp_hbm.at[c, pl.ds(s * 640, 640)])


def _sc_mesh():
  return plsc.VectorSubcoreMesh(
      core_axis_name="c", subcore_axis_name="s", num_cores=NC, num_subcores=NS
  )


def _sc_degrees(dstp):
  k = pl.kernel(
      _deg_body,
      out_type=jax.ShapeDtypeStruct((NC * DEG_ACC,), jnp.float32),
      mesh=_sc_mesh(),
      scratch_types=[
          pltpu.VMEM((DEG_TILE_E,), jnp.int32),
          pltpu.VMEM((DEG_TILE_E,), jnp.float32),
          pltpu.VMEM((DEG_ACC // NS,), jnp.float32),
          pltpu.VMEM_SHARED((DEG_ACC,), jnp.float32),
      ],
  )
  return k(dstp)


def _sc_aggregate(src, dst, hs):
  k = pl.kernel(
      _agg_body,
      out_type=jax.ShapeDtypeStruct((NC, NPAD, D), jnp.float32),
      mesh=_sc_mesh(),
      scratch_types=[
          pltpu.VMEM((2, GROUP, CHUNK_E), jnp.int32),
          pltpu.VMEM((2, GROUP, CHUNK_E), jnp.int32),
          pltpu.VMEM((2, CHUNK_E, D), jnp.float32),
          pltpu.SemaphoreType.DMA((2,)),
          pltpu.SemaphoreType.DMA((2,)),
          pltpu.SemaphoreType.DMA((2,)),
          pltpu.VMEM_SHARED((NPAD, D), jnp.float32),
      ],
  )
  return k(src, dst, hs)


def _dot(a, b):
  return lax.dot_general(
      a, b, (((1,), (0,)), ((), ())),
      precision=lax.Precision.HIGHEST,
      preferred_element_type=jnp.float32,
  )


def _first_body(degt_ref, x_ref, w_ref, dinv_ref, hs_ref):
  deg = degt_ref[:, 0:1] + degt_ref[:, 1:2] + 1.0
  dinv = lax.rsqrt(deg)
  dinv_ref[...] = dinv
  hs_ref[...] = _dot(x_ref[...], w_ref[...]) * dinv


def _mid_body(aggp_ref, hs_ref, dinv_ref, b_ref, g_ref, be_ref, w_ref,
              hs2_ref, *, relu):
  dinv = dinv_ref[...]
  agg = aggp_ref[0, :N] + aggp_ref[1, :N] + hs_ref[...]
  conv = dinv * agg + b_ref[...]
  a = jnp.maximum(conv, 0.0) if relu else conv
  m = jnp.mean(a, axis=0, keepdims=True)
  v = jnp.mean((a - m) ** 2, axis=0, keepdims=True)
  z = (a - m) * lax.rsqrt(v + 1e-5) * g_ref[...] + be_ref[...]
  hs2_ref[...] = _dot(z, w_ref[...]) * dinv


def _final_body(aggp_ref, hs_ref, dinv_ref, b_ref, g_ref, be_ref,
                batch_ref, linw_ref, linb_ref, out_ref):
  dinv = dinv_ref[...]
  agg = aggp_ref[0, :N] + aggp_ref[1, :N] + hs_ref[...]
  conv = dinv * agg + b_ref[...]
  m = jnp.mean(conv, axis=0, keepdims=True)
  v = jnp.mean((conv - m) ** 2, axis=0, keepdims=True)
  z = (conv - m) * lax.rsqrt(v + 1e-5) * g_ref[...] + be_ref[...]
  seg = lax.broadcasted_iota(jnp.int32, (G, N), 0)
  p = (seg == jnp.broadcast_to(batch_ref[...], (G, N))).astype(jnp.float32)
  sums = _dot(p, z)
  counts = jnp.sum(p, axis=1, keepdims=True)
  pooled = sums / jnp.maximum(counts, 1.0)
  out_ref[...] = _dot(pooled, linw_ref[...]) + linb_ref[...]


def _tc_call(body, out_shapes):
  return pl.pallas_call(body, out_shape=out_shapes)


def kernel(x, edge_index, batch, hidden_channels, num_layers,
           W1, b1, g1, be1, W2, b2, g2, be2, W3, b3, g3, be3, linW, linb):
  del hidden_channels, num_layers
  src = edge_index[0]
  dst = edge_index[1]

  padi = jnp.arange(DEG_PAD, dtype=jnp.int32) % (DEG_ACC - N)
  dstp = jnp.concatenate([dst, N + padi])
  src2d = jnp.concatenate(
      [src.reshape(E // 128, 128), padi.reshape(DEG_PAD // 128, 128)])
  dst2d = jnp.concatenate(
      [dst.reshape(E // 128, 128), (N + padi).reshape(DEG_PAD // 128, 128)])

  degp = _sc_degrees(dstp).reshape(NC, DEG_ACC)
  degt = degp[:, :N].T

  dinv, hs1 = _tc_call(
      _first_body,
      (jax.ShapeDtypeStruct((N, 1), jnp.float32),
       jax.ShapeDtypeStruct((N, D), jnp.float32)),
  )(degt, x, W1)

  b1r, g1r, be1r = b1.reshape(1, D), g1.reshape(1, D), be1.reshape(1, D)
  b2r, g2r, be2r = b2.reshape(1, D), g2.reshape(1, D), be2.reshape(1, D)
  b3r, g3r, be3r = b3.reshape(1, D), g3.reshape(1, D), be3.reshape(1, D)

  agg1p = _sc_aggregate(src2d, dst2d, hs1)
  hs2 = _tc_call(
      functools.partial(_mid_body, relu=True),
      jax.ShapeDtypeStruct((N, D), jnp.float32),
  )(agg1p, hs1, dinv, b1r, g1r, be1r, W2)

  agg2p = _sc_aggregate(src2d, dst2d, hs2)
  hs3 = _tc_call(
      functools.partial(_mid_body, relu=True),
      jax.ShapeDtypeStruct((N, D), jnp.float32),
  )(agg2p, hs2, dinv, b2r, g2r, be2r, W3)

  agg3p = _sc_aggregate(src2d, dst2d, hs3)
  out = _tc_call(
      _final_body, jax.ShapeDtypeStruct((G, C), jnp.float32),
  )(agg3p, hs3, dinv, b3r, g3r, be3r,
    batch.reshape(1, N), linW, linb.reshape(1, C))
  return out
